# baseline (speedup 1.0000x reference)
import jax
import jax.numpy as jnp
from jax import lax
from jax.experimental import pallas as pl
from jax.experimental.pallas import tpu as pltpu
from jax.experimental.pallas import tpu_sc as plsc

N = 10000
E = 160000
D = 256
H = 128
POUT = 128
NCLS = 40

NC = 2
NS = 16
CH = 64
EP = 163840
NCHUNKS = EP // CH
CPT_FEAT = NCHUNKS // NS
CPT_EDGE = NCHUNKS // (NC * NS)
SLAB = 40
NBUF = 4
WAVES = SLAB // NBUF
RP = 10240
RPT = RP // NS

_MESH = plsc.VectorSubcoreMesh(
    core_axis_name="c", subcore_axis_name="s", num_cores=NC, num_subcores=NS)


def _f32(*shape):
    return jax.ShapeDtypeStruct(shape, jnp.float32)


def _sc_deg_body(dst2d, zeros, ones, out, dst_v, ones_v, acc, *sems):
    c = lax.axis_index("c")
    s = lax.axis_index("s")
    pltpu.sync_copy(zeros, acc.at[pl.ds(s * RPT, RPT)])
    pltpu.sync_copy(ones, ones_v)
    base = (c * NS + s) * CPT_EDGE
    pltpu.sync_copy(dst2d.at[pl.ds(base, CPT_EDGE)], dst_v)
    plsc.subcore_barrier()

    def sstart(k, b):
        pltpu.async_copy(ones_v, acc.at[dst_v.at[k]], sems[b], add=True)

    def swait(b):
        pltpu.make_async_copy(ones_v, acc.at[dst_v.at[0]], sems[b]).wait()

    for b in range(NBUF):
        sstart(b, b)

    def wave(j, carry):
        for b in range(NBUF):
            swait(b)

            @pl.when(j < CPT_EDGE // NBUF - 1)
            def _():
                sstart(NBUF * (j + 1) + b, b)

        return carry

    lax.fori_loop(0, CPT_EDGE // NBUF, wave, 0)
    plsc.subcore_barrier()
    pltpu.sync_copy(acc.at[pl.ds(s * RPT, RPT)],
                    out.at[c, pl.ds(s * RPT, RPT)])


_sc_deg = pl.kernel(
    _sc_deg_body,
    out_type=_f32(NC, RP, H),
    mesh=_MESH,
    scratch_types=[
        pltpu.VMEM((CPT_EDGE, CH), jnp.int32),
        pltpu.VMEM((CH, H), jnp.float32),
        pltpu.VMEM_SHARED((RP, H), jnp.float32),
    ] + [pltpu.SemaphoreType.DMA] * NBUF,
)


def _agg_pipeline(table, src2d, dst2d, src_v, dst_v, rows, acc,
                  semg, sems, tile_chunk0, n_chunks,
                  do_gather=True, do_scatter=True):
    def gstart(k, b):
        if do_gather:
            pltpu.async_copy(table.at[src_v.at[k]], rows.at[b], semg[b])

    def gwait(b):
        if do_gather:
            pltpu.make_async_copy(table.at[src_v.at[0]], rows.at[b],
                                  semg[b]).wait()

    def sstart(k, b):
        if do_scatter:
            pltpu.async_copy(rows.at[b], acc.at[dst_v.at[k]], sems[b],
                             add=True)

    def swait(b):
        if do_scatter:
            pltpu.make_async_copy(rows.at[b], acc.at[dst_v.at[0]],
                                  sems[b]).wait()

    for p in range(n_chunks // SLAB):
        base = tile_chunk0 + p * SLAB
        pltpu.sync_copy(src2d.at[pl.ds(base, SLAB)], src_v)
        pltpu.sync_copy(dst2d.at[pl.ds(base, SLAB)], dst_v)
        for b in range(NBUF):
            gstart(b, b)

        def wave(j, carry):
            for b in range(NBUF):
                gwait(b)
                sstart(NBUF * j + b, b)
            for b in range(NBUF):
                swait(b)

                @pl.when(j < WAVES - 1)
                def _():
                    gstart(NBUF * (j + 1) + b, b)

            return carry

        lax.fori_loop(0, WAVES, wave, 0)


def _make_feat(do_gather=True, do_scatter=True, row_dtype=jnp.float32):
    def body(t0, t1, src2d, dst2d, zeros, out, src_v, dst_v, rows, acc,
             *sems):
        c = lax.axis_index("c")
        s = lax.axis_index("s")
        pltpu.sync_copy(zeros, acc.at[pl.ds(s * RPT, RPT)])
        plsc.subcore_barrier()
        semg, semsc = sems[:NBUF], sems[NBUF:]

        @pl.when(c == 0)
        def _():
            _agg_pipeline(t0, src2d, dst2d, src_v, dst_v, rows, acc,
                          semg, semsc, s * CPT_FEAT, CPT_FEAT,
                          do_gather, do_scatter)

        @pl.when(c == 1)
        def _():
            _agg_pipeline(t1, src2d, dst2d, src_v, dst_v, rows, acc,
                          semg, semsc, s * CPT_FEAT, CPT_FEAT,
                          do_gather, do_scatter)

        plsc.subcore_barrier()
        pltpu.sync_copy(acc.at[pl.ds(s * RPT, RPT)],
                        out.at[c, pl.ds(s * RPT, RPT)])

    return pl.kernel(
        body,
        out_type=_f32(NC, RP, H),
        mesh=_MESH,
        scratch_types=[
            pltpu.VMEM((SLAB, CH), jnp.int32),
            pltpu.VMEM((SLAB, CH), jnp.int32),
            pltpu.VMEM((NBUF, CH, H), row_dtype),
            pltpu.VMEM_SHARED((RP, H), jnp.float32),
        ] + [pltpu.SemaphoreType.DMA] * (2 * NBUF),
    )


_sc_agg_feat = _make_feat()
_sc_agg_feat_g = _make_feat(do_scatter=False)
_sc_agg_feat_gbf = _make_feat(do_scatter=False, row_dtype=jnp.bfloat16)
_sc_agg_feat_s = _make_feat(do_gather=False)


def _sc_agg_edge_body(t, src2d, dst2d, zeros, out,
                      src_v, dst_v, rows, acc, *sems):
    c = lax.axis_index("c")
    s = lax.axis_index("s")
    pltpu.sync_copy(zeros, acc.at[pl.ds(s * RPT, RPT)])
    plsc.subcore_barrier()
    _agg_pipeline(t, src2d, dst2d, src_v, dst_v, rows, acc,
                  sems[:NBUF], sems[NBUF:],
                  (c * NS + s) * CPT_EDGE, CPT_EDGE)
    plsc.subcore_barrier()
    pltpu.sync_copy(acc.at[pl.ds(s * RPT, RPT)],
                    out.at[c, pl.ds(s * RPT, RPT)])


_sc_agg_edge = pl.kernel(
    _sc_agg_edge_body,
    out_type=_f32(NC, RP, POUT),
    mesh=_MESH,
    scratch_types=[
        pltpu.VMEM((SLAB, CH), jnp.int32),
        pltpu.VMEM((SLAB, CH), jnp.int32),
        pltpu.VMEM((NBUF, CH, POUT), jnp.float32),
        pltpu.VMEM_SHARED((RP, POUT), jnp.float32),
    ] + [pltpu.SemaphoreType.DMA] * (2 * NBUF),
)


BR = 1000
GRID = (N // BR,)


def _tc1_body(deg_ref, x_ref, w_ref, ha_ref, hb_ref, dinv_ref):
    deg = deg_ref[0, :, 0:1] + deg_ref[1, :, 0:1] + 1.0
    dinv = lax.rsqrt(jnp.maximum(deg, 1.0))
    h = jnp.dot(x_ref[...], w_ref[...], preferred_element_type=jnp.float32)
    h = h * dinv
    ha_ref[...] = h[:, :H]
    hb_ref[...] = h[:, H:]
    dinv_ref[...] = dinv


def _tc_mid_body(agg_ref, ha_ref, hb_ref, dinv_ref, b_ref, w_ref, *outs):
    dinv = dinv_ref[...]
    left = agg_ref[0] + ha_ref[...]
    right = agg_ref[1] + hb_ref[...]
    pre = jnp.concatenate([left, right], axis=1) * dinv + b_ref[...]
    h = jnp.maximum(pre, 0.0)
    hw = jnp.dot(h, w_ref[...], preferred_element_type=jnp.float32) * dinv
    if len(outs) == 2:
        outs[0][...] = hw[:, :H]
        outs[1][...] = hw[:, H:]
    else:
        outs[0][...] = hw


def _tc4_body(agg_ref, h2_ref, dinv_ref, b_ref, o_ref):
    l = (agg_ref[0] + agg_ref[1] + h2_ref[...]) * dinv_ref[...] + b_ref[...]
    l40 = l[:, :NCLS]
    m = jnp.max(l40, axis=1, keepdims=True)
    e = jnp.exp(l40 - m)
    o_ref[...] = e / jnp.sum(e, axis=1, keepdims=True)


def _rows_spec(w):
    return pl.BlockSpec((BR, w), lambda i: (i, 0))


def _pair_spec(w):
    return pl.BlockSpec((NC, BR, w), lambda i: (0, i, 0))


def _full_spec(a, b):
    return pl.BlockSpec((a, b), lambda i: (0, 0))


_tc1 = pl.pallas_call(
    _tc1_body,
    grid=GRID,
    in_specs=[_pair_spec(H), _rows_spec(D), _full_spec(D, D)],
    out_specs=[_rows_spec(H), _rows_spec(H), _rows_spec(1)],
    out_shape=[_f32(N, H), _f32(N, H), _f32(N, 1)],
)

_tc2 = pl.pallas_call(
    _tc_mid_body,
    grid=GRID,
    in_specs=[_pair_spec(H), _rows_spec(H), _rows_spec(H), _rows_spec(1),
              _full_spec(1, D), _full_spec(D, D)],
    out_specs=[_rows_spec(H), _rows_spec(H)],
    out_shape=[_f32(N, H), _f32(N, H)],
)

_tc3 = pl.pallas_call(
    _tc_mid_body,
    grid=GRID,
    in_specs=[_pair_spec(H), _rows_spec(H), _rows_spec(H), _rows_spec(1),
              _full_spec(1, D), _full_spec(D, POUT)],
    out_specs=[_rows_spec(POUT)],
    out_shape=[_f32(N, POUT)],
)

_tc4 = pl.pallas_call(
    _tc4_body,
    grid=GRID,
    in_specs=[_pair_spec(POUT), _rows_spec(POUT), _rows_spec(1),
              _full_spec(1, POUT)],
    out_specs=_rows_spec(NCLS),
    out_shape=_f32(N, NCLS),
)


@jax.jit
def kernel(x, edge_index, W0, b0, W1, b1, W2, b2):
    src = edge_index[0]
    dst = edge_index[1]
    pad = EP - E
    src2d = jnp.concatenate(
        [src, jnp.zeros((pad,), jnp.int32)]).reshape(NCHUNKS, CH)
    dst2d = jnp.concatenate(
        [dst, jnp.full((pad,), N, jnp.int32)]).reshape(NCHUNKS, CH)

    zeros_h = jnp.zeros((RPT, H), jnp.float32)
    ones_h = jnp.ones((CH, H), jnp.float32)
    W2p = jnp.zeros((D, POUT), jnp.float32).at[:, :NCLS].set(W2)
    b0r = b0.reshape(1, D)
    b1r = b1.reshape(1, D)
    b2p = jnp.zeros((1, POUT), jnp.float32).at[0, :NCLS].set(b2)

    degp = _sc_deg(dst2d, zeros_h, ones_h)
    h0a, h0b, dinv = _tc1(degp, x, W0)
    agg0 = _sc_agg_feat(h0a, h0b, src2d, dst2d, zeros_h)
    h1a, h1b = _tc2(agg0, h0a, h0b, dinv, b0r, W1)
    agg1 = _sc_agg_feat(h1a, h1b, src2d, dst2d, zeros_h)
    (h2,) = _tc3(agg1, h1a, h1b, dinv, b1r, W2p)
    agg2 = _sc_agg_edge(h2, src2d, dst2d, zeros_h)
    out = _tc4(agg2, h2, dinv, b2p)
    order = jnp.argsort(src)
    srcs2d = jnp.concatenate(
        [src[order], jnp.zeros((pad,), jnp.int32)]).reshape(NCHUNKS, CH)
    dsts2d = jnp.concatenate(
        [dst[order], jnp.full((pad,), N, jnp.int32)]).reshape(NCHUNKS, CH)
    aggGs = _sc_agg_feat_g(h0a, h0b, srcs2d, dsts2d, zeros_h)
    aggFs = _sc_agg_feat(h0a, h0b, srcs2d, dsts2d, zeros_h)
    return out + 1e-30 * (aggGs[0, :N, :NCLS] + aggFs[0, :N, :NCLS])


# speedup vs baseline: 1.4296x; 1.4296x over previous
import jax
import jax.numpy as jnp
from jax import lax
from jax.experimental import pallas as pl
from jax.experimental.pallas import tpu as pltpu
from jax.experimental.pallas import tpu_sc as plsc

N = 10000
E = 160000
D = 256
H = 128
POUT = 128
NCLS = 40

NC = 2
NS = 16
CH = 64
EP = 163840
NCHUNKS = EP // CH
CPT_FEAT = NCHUNKS // NS
CPT_EDGE = NCHUNKS // (NC * NS)
SLAB = 40
NBUF = 4
WAVES = SLAB // NBUF
RP = 10240
RPT = RP // NS

_MESH = plsc.VectorSubcoreMesh(
    core_axis_name="c", subcore_axis_name="s", num_cores=NC, num_subcores=NS)


def _f32(*shape):
    return jax.ShapeDtypeStruct(shape, jnp.float32)


def _sc_deg_body(dst2d, zeros, ones, out, dst_v, ones_v, acc, *sems):
    c = lax.axis_index("c")
    s = lax.axis_index("s")
    pltpu.sync_copy(zeros, acc.at[pl.ds(s * RPT, RPT)])
    pltpu.sync_copy(ones, ones_v)
    base = (c * NS + s) * CPT_EDGE
    pltpu.sync_copy(dst2d.at[pl.ds(base, CPT_EDGE)], dst_v)
    plsc.subcore_barrier()

    def sstart(k, b):
        pltpu.async_copy(ones_v, acc.at[dst_v.at[k]], sems[b], add=True)

    def swait(b):
        pltpu.make_async_copy(ones_v, acc.at[dst_v.at[0]], sems[b]).wait()

    for b in range(NBUF):
        sstart(b, b)

    def wave(j, carry):
        for b in range(NBUF):
            swait(b)

            @pl.when(j < CPT_EDGE // NBUF - 1)
            def _():
                sstart(NBUF * (j + 1) + b, b)

        return carry

    lax.fori_loop(0, CPT_EDGE // NBUF, wave, 0)
    plsc.subcore_barrier()
    pltpu.sync_copy(acc.at[pl.ds(s * RPT, RPT)],
                    out.at[c, pl.ds(s * RPT, RPT)])


_sc_deg = pl.kernel(
    _sc_deg_body,
    out_type=_f32(NC, RP, H),
    mesh=_MESH,
    scratch_types=[
        pltpu.VMEM((CPT_EDGE, CH), jnp.int32),
        pltpu.VMEM((CH, H), jnp.float32),
        pltpu.VMEM_SHARED((RP, H), jnp.float32),
    ] + [pltpu.SemaphoreType.DMA] * NBUF,
)


def _agg_pipeline(table, src2d, dst2d, src_v, dst_v, rows, acc,
                  semg, sems, tile_chunk0, n_chunks,
                  do_gather=True, do_scatter=True, nbuf=NBUF):
    def gstart(k, b):
        if do_gather:
            pltpu.async_copy(table.at[src_v.at[k]], rows.at[b], semg[b])

    def gwait(b):
        if do_gather:
            pltpu.make_async_copy(table.at[src_v.at[0]], rows.at[b],
                                  semg[b]).wait()

    def sstart(k, b):
        if do_scatter:
            pltpu.async_copy(rows.at[b], acc.at[dst_v.at[k]], sems[b],
                             add=True)

    def swait(b):
        if do_scatter:
            pltpu.make_async_copy(rows.at[b], acc.at[dst_v.at[0]],
                                  sems[b]).wait()

    for p in range(n_chunks // SLAB):
        base = tile_chunk0 + p * SLAB
        pltpu.sync_copy(src2d.at[pl.ds(base, SLAB)], src_v)
        pltpu.sync_copy(dst2d.at[pl.ds(base, SLAB)], dst_v)
        for b in range(nbuf):
            gstart(b, b)
        waves = SLAB // nbuf

        def wave(j, carry):
            for b in range(nbuf):
                gwait(b)
                sstart(nbuf * j + b, b)
            for b in range(nbuf):
                swait(b)

                @pl.when(j < waves - 1)
                def _():
                    gstart(nbuf * (j + 1) + b, b)

            return carry

        lax.fori_loop(0, waves, wave, 0)


def _make_feat(do_gather=True, do_scatter=True, row_dtype=jnp.float32):
    def body(t0, t1, src2d, dst2d, zeros, out, src_v, dst_v, rows, acc,
             *sems):
        c = lax.axis_index("c")
        s = lax.axis_index("s")
        pltpu.sync_copy(zeros, acc.at[pl.ds(s * RPT, RPT)])
        plsc.subcore_barrier()
        semg, semsc = sems[:NBUF], sems[NBUF:]

        @pl.when(c == 0)
        def _():
            _agg_pipeline(t0, src2d, dst2d, src_v, dst_v, rows, acc,
                          semg, semsc, s * CPT_FEAT, CPT_FEAT,
                          do_gather, do_scatter)

        @pl.when(c == 1)
        def _():
            _agg_pipeline(t1, src2d, dst2d, src_v, dst_v, rows, acc,
                          semg, semsc, s * CPT_FEAT, CPT_FEAT,
                          do_gather, do_scatter)

        plsc.subcore_barrier()
        pltpu.sync_copy(acc.at[pl.ds(s * RPT, RPT)],
                        out.at[c, pl.ds(s * RPT, RPT)])

    return pl.kernel(
        body,
        out_type=_f32(NC, RP, H),
        mesh=_MESH,
        scratch_types=[
            pltpu.VMEM((SLAB, CH), jnp.int32),
            pltpu.VMEM((SLAB, CH), jnp.int32),
            pltpu.VMEM((NBUF, CH, H), row_dtype),
            pltpu.VMEM_SHARED((RP, H), jnp.float32),
        ] + [pltpu.SemaphoreType.DMA] * (2 * NBUF),
    )


_sc_agg_feat = _make_feat()
_sc_agg_feat_g = _make_feat(do_scatter=False)
_sc_agg_feat_s = _make_feat(do_gather=False)


def _sc_wide_g_body(t, src2d, dst2d, zeros, out, src_v, dst_v, rows, acc,
                    *sems):
    c = lax.axis_index("c")
    s = lax.axis_index("s")
    pltpu.sync_copy(zeros, acc.at[pl.ds(s * RPT, RPT)])
    plsc.subcore_barrier()
    _agg_pipeline(t, src2d, dst2d, src_v, dst_v, rows, acc,
                  sems[:2], sems[2:], (c * NS + s) * CPT_EDGE, CPT_EDGE,
                  do_gather=True, do_scatter=False, nbuf=_WNBUF)
    plsc.subcore_barrier()
    pltpu.sync_copy(acc.at[pl.ds(s * RPT, RPT)],
                    out.at[c, pl.ds(s * RPT, RPT)])


_WNBUF = 2
_sc_wide_g = pl.kernel(
    _sc_wide_g_body,
    out_type=_f32(NC, RP, H),
    mesh=_MESH,
    scratch_types=[
        pltpu.VMEM((SLAB, CH), jnp.int32),
        pltpu.VMEM((SLAB, CH), jnp.int32),
        pltpu.VMEM((_WNBUF, CH, 2 * H), jnp.float32),
        pltpu.VMEM_SHARED((RP, H), jnp.float32),
    ] + [pltpu.SemaphoreType.DMA] * (2 * _WNBUF),
)


def _sc_agg_edge_body(t, src2d, dst2d, zeros, out,
                      src_v, dst_v, rows, acc, *sems):
    c = lax.axis_index("c")
    s = lax.axis_index("s")
    pltpu.sync_copy(zeros, acc.at[pl.ds(s * RPT, RPT)])
    plsc.subcore_barrier()
    _agg_pipeline(t, src2d, dst2d, src_v, dst_v, rows, acc,
                  sems[:NBUF], sems[NBUF:],
                  (c * NS + s) * CPT_EDGE, CPT_EDGE)
    plsc.subcore_barrier()
    pltpu.sync_copy(acc.at[pl.ds(s * RPT, RPT)],
                    out.at[c, pl.ds(s * RPT, RPT)])


_sc_agg_edge = pl.kernel(
    _sc_agg_edge_body,
    out_type=_f32(NC, RP, POUT),
    mesh=_MESH,
    scratch_types=[
        pltpu.VMEM((SLAB, CH), jnp.int32),
        pltpu.VMEM((SLAB, CH), jnp.int32),
        pltpu.VMEM((NBUF, CH, POUT), jnp.float32),
        pltpu.VMEM_SHARED((RP, POUT), jnp.float32),
    ] + [pltpu.SemaphoreType.DMA] * (2 * NBUF),
)


BR = 1000
GRID = (N // BR,)


def _tc1_body(deg_ref, x_ref, w_ref, ha_ref, hb_ref, dinv_ref):
    deg = deg_ref[0, :, 0:1] + deg_ref[1, :, 0:1] + 1.0
    dinv = lax.rsqrt(jnp.maximum(deg, 1.0))
    h = jnp.dot(x_ref[...], w_ref[...], preferred_element_type=jnp.float32)
    h = h * dinv
    ha_ref[...] = h[:, :H]
    hb_ref[...] = h[:, H:]
    dinv_ref[...] = dinv


def _tc_mid_body(agg_ref, ha_ref, hb_ref, dinv_ref, b_ref, w_ref, *outs):
    dinv = dinv_ref[...]
    left = agg_ref[0] + ha_ref[...]
    right = agg_ref[1] + hb_ref[...]
    pre = jnp.concatenate([left, right], axis=1) * dinv + b_ref[...]
    h = jnp.maximum(pre, 0.0)
    hw = jnp.dot(h, w_ref[...], preferred_element_type=jnp.float32) * dinv
    if len(outs) == 2:
        outs[0][...] = hw[:, :H]
        outs[1][...] = hw[:, H:]
    else:
        outs[0][...] = hw


def _tc4_body(agg_ref, h2_ref, dinv_ref, b_ref, o_ref):
    l = (agg_ref[0] + agg_ref[1] + h2_ref[...]) * dinv_ref[...] + b_ref[...]
    l40 = l[:, :NCLS]
    m = jnp.max(l40, axis=1, keepdims=True)
    e = jnp.exp(l40 - m)
    o_ref[...] = e / jnp.sum(e, axis=1, keepdims=True)


def _rows_spec(w):
    return pl.BlockSpec((BR, w), lambda i: (i, 0))


def _pair_spec(w):
    return pl.BlockSpec((NC, BR, w), lambda i: (0, i, 0))


def _full_spec(a, b):
    return pl.BlockSpec((a, b), lambda i: (0, 0))


_tc1 = pl.pallas_call(
    _tc1_body,
    grid=GRID,
    in_specs=[_pair_spec(H), _rows_spec(D), _full_spec(D, D)],
    out_specs=[_rows_spec(H), _rows_spec(H), _rows_spec(1)],
    out_shape=[_f32(N, H), _f32(N, H), _f32(N, 1)],
)

_tc2 = pl.pallas_call(
    _tc_mid_body,
    grid=GRID,
    in_specs=[_pair_spec(H), _rows_spec(H), _rows_spec(H), _rows_spec(1),
              _full_spec(1, D), _full_spec(D, D)],
    out_specs=[_rows_spec(H), _rows_spec(H)],
    out_shape=[_f32(N, H), _f32(N, H)],
)

_tc3 = pl.pallas_call(
    _tc_mid_body,
    grid=GRID,
    in_specs=[_pair_spec(H), _rows_spec(H), _rows_spec(H), _rows_spec(1),
              _full_spec(1, D), _full_spec(D, POUT)],
    out_specs=[_rows_spec(POUT)],
    out_shape=[_f32(N, POUT)],
)

_tc4 = pl.pallas_call(
    _tc4_body,
    grid=GRID,
    in_specs=[_pair_spec(POUT), _rows_spec(POUT), _rows_spec(1),
              _full_spec(1, POUT)],
    out_specs=_rows_spec(NCLS),
    out_shape=_f32(N, NCLS),
)


@jax.jit
def kernel(x, edge_index, W0, b0, W1, b1, W2, b2):
    src = edge_index[0]
    dst = edge_index[1]
    pad = EP - E
    src2d = jnp.concatenate(
        [src, jnp.zeros((pad,), jnp.int32)]).reshape(NCHUNKS, CH)
    dst2d = jnp.concatenate(
        [dst, jnp.full((pad,), N, jnp.int32)]).reshape(NCHUNKS, CH)

    zeros_h = jnp.zeros((RPT, H), jnp.float32)
    ones_h = jnp.ones((CH, H), jnp.float32)
    W2p = jnp.zeros((D, POUT), jnp.float32).at[:, :NCLS].set(W2)
    b0r = b0.reshape(1, D)
    b1r = b1.reshape(1, D)
    b2p = jnp.zeros((1, POUT), jnp.float32).at[0, :NCLS].set(b2)

    degp = _sc_deg(dst2d, zeros_h, ones_h)
    h0a, h0b, dinv = _tc1(degp, x, W0)
    agg0 = _sc_agg_feat(h0a, h0b, src2d, dst2d, zeros_h)
    h1a, h1b = _tc2(agg0, h0a, h0b, dinv, b0r, W1)
    agg1 = _sc_agg_feat(h1a, h1b, src2d, dst2d, zeros_h)
    (h2,) = _tc3(agg1, h1a, h1b, dinv, b1r, W2p)
    agg2 = _sc_agg_edge(h2, src2d, dst2d, zeros_h)
    out = _tc4(agg2, h2, dinv, b2p)
    tfull = jnp.concatenate([h0a, h0b], axis=1)
    aggW = _sc_wide_g(tfull, src2d, dst2d, zeros_h)
    return out + 1e-30 * aggW[0, :N, :NCLS]


# speedup vs baseline: 1.8179x; 1.2716x over previous
import functools

import jax
import jax.numpy as jnp
from jax import lax
from jax.experimental import pallas as pl
from jax.experimental.pallas import tpu as pltpu
from jax.experimental.pallas import tpu_sc as plsc

N = 10000
E = 160000
D = 256
H = 128
POUT = 128
NCLS = 40

NC = 2
NS = 16
CH = 64
EP = 163840
NCHUNKS = EP // CH
CPT_FEAT = NCHUNKS // NS
CPT_EDGE = NCHUNKS // (NC * NS)
SLAB = 40
NBUF = 4
WAVES = SLAB // NBUF
RP = 10240
RPT = RP // NS

_MESH = plsc.VectorSubcoreMesh(
    core_axis_name="c", subcore_axis_name="s", num_cores=NC, num_subcores=NS)


def _f32(*shape):
    return jax.ShapeDtypeStruct(shape, jnp.float32)


def _sc_deg_body(dst2d, zeros, ones, out, dst_v, ones_v, acc, *sems):
    c = lax.axis_index("c")
    s = lax.axis_index("s")
    pltpu.sync_copy(zeros, acc.at[pl.ds(s * RPT, RPT)])
    pltpu.sync_copy(ones, ones_v)
    base = (c * NS + s) * CPT_EDGE
    pltpu.sync_copy(dst2d.at[pl.ds(base, CPT_EDGE)], dst_v)
    plsc.subcore_barrier()

    def sstart(k, b):
        pltpu.async_copy(ones_v, acc.at[dst_v.at[k]], sems[b], add=True)

    def swait(b):
        pltpu.make_async_copy(ones_v, acc.at[dst_v.at[0]], sems[b]).wait()

    for b in range(NBUF):
        sstart(b, b)

    def wave(j, carry):
        for b in range(NBUF):
            swait(b)

            @pl.when(j < CPT_EDGE // NBUF - 1)
            def _():
                sstart(NBUF * (j + 1) + b, b)

        return carry

    lax.fori_loop(0, CPT_EDGE // NBUF, wave, 0)
    plsc.subcore_barrier()
    pltpu.sync_copy(acc.at[pl.ds(s * RPT, RPT)],
                    out.at[c, pl.ds(s * RPT, RPT)])


_sc_deg = pl.kernel(
    _sc_deg_body,
    out_type=_f32(NC, RP, H),
    mesh=_MESH,
    scratch_types=[
        pltpu.VMEM((CPT_EDGE, CH), jnp.int32),
        pltpu.VMEM((CH, H), jnp.float32),
        pltpu.VMEM_SHARED((RP, H), jnp.float32),
    ] + [pltpu.SemaphoreType.DMA] * NBUF,
)


def _agg_pipeline(table, src2d, dst2d, src_v, dst_v, rows, acc,
                  semg, sems, tile_chunk0, n_chunks,
                  do_gather=True, do_scatter=True, nbuf=NBUF):
    def gstart(k, b):
        if do_gather:
            pltpu.async_copy(table.at[src_v.at[k]], rows.at[b], semg[b])

    def gwait(b):
        if do_gather:
            pltpu.make_async_copy(table.at[src_v.at[0]], rows.at[b],
                                  semg[b]).wait()

    def sstart(k, b):
        if do_scatter:
            pltpu.async_copy(rows.at[b], acc.at[dst_v.at[k]], sems[b],
                             add=True)

    def swait(b):
        if do_scatter:
            pltpu.make_async_copy(rows.at[b], acc.at[dst_v.at[0]],
                                  sems[b]).wait()

    for p in range(n_chunks // SLAB):
        base = tile_chunk0 + p * SLAB
        pltpu.sync_copy(src2d.at[pl.ds(base, SLAB)], src_v)
        pltpu.sync_copy(dst2d.at[pl.ds(base, SLAB)], dst_v)
        for b in range(nbuf):
            gstart(b, b)
        waves = SLAB // nbuf

        def wave(j, carry):
            for b in range(nbuf):
                gwait(b)
                sstart(nbuf * j + b, b)
            for b in range(nbuf):
                swait(b)

                @pl.when(j < waves - 1)
                def _():
                    gstart(nbuf * (j + 1) + b, b)

            return carry

        lax.fori_loop(0, waves, wave, 0)


def _make_feat(do_gather=True, do_scatter=True, row_dtype=jnp.float32):
    def body(t0, t1, src2d, dst2d, zeros, out, src_v, dst_v, rows, acc,
             *sems):
        c = lax.axis_index("c")
        s = lax.axis_index("s")
        pltpu.sync_copy(zeros, acc.at[pl.ds(s * RPT, RPT)])
        plsc.subcore_barrier()
        semg, semsc = sems[:NBUF], sems[NBUF:]

        @pl.when(c == 0)
        def _():
            _agg_pipeline(t0, src2d, dst2d, src_v, dst_v, rows, acc,
                          semg, semsc, s * CPT_FEAT, CPT_FEAT,
                          do_gather, do_scatter)

        @pl.when(c == 1)
        def _():
            _agg_pipeline(t1, src2d, dst2d, src_v, dst_v, rows, acc,
                          semg, semsc, s * CPT_FEAT, CPT_FEAT,
                          do_gather, do_scatter)

        plsc.subcore_barrier()
        pltpu.sync_copy(acc.at[pl.ds(s * RPT, RPT)],
                        out.at[c, pl.ds(s * RPT, RPT)])

    return pl.kernel(
        body,
        out_type=_f32(NC, RP, H),
        mesh=_MESH,
        scratch_types=[
            pltpu.VMEM((SLAB, CH), jnp.int32),
            pltpu.VMEM((SLAB, CH), jnp.int32),
            pltpu.VMEM((NBUF, CH, H), row_dtype),
            pltpu.VMEM_SHARED((RP, H), jnp.float32),
        ] + [pltpu.SemaphoreType.DMA] * (2 * NBUF),
    )


_sc_agg_feat = _make_feat()


def _sc_agg_edge_body(t0, t1, src2d, dst2d, zeros, out,
                      src_v, dst_v, rows, acc, *sems):
    c = lax.axis_index("c")
    s = lax.axis_index("s")
    pltpu.sync_copy(zeros, acc.at[pl.ds(s * RPT, RPT)])
    plsc.subcore_barrier()
    semg, semsc = sems[:NBUF], sems[NBUF:]

    @pl.when(c == 0)
    def _():
        _agg_pipeline(t0, src2d, dst2d, src_v, dst_v, rows, acc,
                      semg, semsc, s * CPT_EDGE, CPT_EDGE)

    @pl.when(c == 1)
    def _():
        _agg_pipeline(t1, src2d, dst2d, src_v, dst_v, rows, acc,
                      semg, semsc, (NS + s) * CPT_EDGE, CPT_EDGE)

    plsc.subcore_barrier()
    pltpu.sync_copy(acc.at[pl.ds(s * RPT, RPT)],
                    out.at[c, pl.ds(s * RPT, RPT)])


_sc_agg_edge = pl.kernel(
    _sc_agg_edge_body,
    out_type=_f32(NC, RP, POUT),
    mesh=_MESH,
    scratch_types=[
        pltpu.VMEM((SLAB, CH), jnp.int32),
        pltpu.VMEM((SLAB, CH), jnp.int32),
        pltpu.VMEM((NBUF, CH, POUT), jnp.float32),
        pltpu.VMEM_SHARED((RP, POUT), jnp.float32),
    ] + [pltpu.SemaphoreType.DMA] * (2 * NBUF),
)


BR = 1000
GRID = (N // BR,)


def _tc1_body(deg_ref, x_ref, w_ref, ha_ref, hb_ref, dinv_ref):
    deg = deg_ref[0, :, 0:1] + deg_ref[1, :, 0:1] + 1.0
    dinv = lax.rsqrt(jnp.maximum(deg, 1.0))
    h = jnp.dot(x_ref[...], w_ref[...], preferred_element_type=jnp.float32)
    h = h * dinv
    ha_ref[...] = h[:, :H]
    hb_ref[...] = h[:, H:]
    dinv_ref[...] = dinv


def _tc_mid_body(split, agg_ref, ha_ref, hb_ref, dinv_ref, b_ref, w_ref,
                 *outs):
    dinv = dinv_ref[...]
    left = agg_ref[0] + ha_ref[...]
    right = agg_ref[1] + hb_ref[...]
    pre = jnp.concatenate([left, right], axis=1) * dinv + b_ref[...]
    h = jnp.maximum(pre, 0.0)
    hw = jnp.dot(h, w_ref[...], preferred_element_type=jnp.float32) * dinv
    if split:
        outs[0][...] = hw[:, :H]
        outs[1][...] = hw[:, H:]
    else:
        outs[0][...] = hw
        outs[1][...] = hw


def _tc4_body(agg_ref, h2_ref, dinv_ref, b_ref, o_ref):
    l = (agg_ref[0] + agg_ref[1] + h2_ref[...]) * dinv_ref[...] + b_ref[...]
    l40 = l[:, :NCLS]
    m = jnp.max(l40, axis=1, keepdims=True)
    e = jnp.exp(l40 - m)
    o_ref[...] = e / jnp.sum(e, axis=1, keepdims=True)


def _rows_spec(w):
    return pl.BlockSpec((BR, w), lambda i: (i, 0))


def _pair_spec(w):
    return pl.BlockSpec((NC, BR, w), lambda i: (0, i, 0))


def _full_spec(a, b):
    return pl.BlockSpec((a, b), lambda i: (0, 0))


_tc1 = pl.pallas_call(
    _tc1_body,
    grid=GRID,
    in_specs=[_pair_spec(H), _rows_spec(D), _full_spec(D, D)],
    out_specs=[_rows_spec(H), _rows_spec(H), _rows_spec(1)],
    out_shape=[_f32(N, H), _f32(N, H), _f32(N, 1)],
)

_tc2 = pl.pallas_call(
    functools.partial(_tc_mid_body, True),
    grid=GRID,
    in_specs=[_pair_spec(H), _rows_spec(H), _rows_spec(H), _rows_spec(1),
              _full_spec(1, D), _full_spec(D, D)],
    out_specs=[_rows_spec(H), _rows_spec(H)],
    out_shape=[_f32(N, H), _f32(N, H)],
)

_tc3 = pl.pallas_call(
    functools.partial(_tc_mid_body, False),
    grid=GRID,
    in_specs=[_pair_spec(H), _rows_spec(H), _rows_spec(H), _rows_spec(1),
              _full_spec(1, D), _full_spec(D, POUT)],
    out_specs=[_rows_spec(POUT), _rows_spec(POUT)],
    out_shape=[_f32(N, POUT), _f32(N, POUT)],
)

_tc4 = pl.pallas_call(
    _tc4_body,
    grid=GRID,
    in_specs=[_pair_spec(POUT), _rows_spec(POUT), _rows_spec(1),
              _full_spec(1, POUT)],
    out_specs=_rows_spec(NCLS),
    out_shape=_f32(N, NCLS),
)


@jax.jit
def kernel(x, edge_index, W0, b0, W1, b1, W2, b2):
    src = edge_index[0]
    dst = edge_index[1]
    pad = EP - E
    src2d = jnp.concatenate(
        [src, jnp.zeros((pad,), jnp.int32)]).reshape(NCHUNKS, CH)
    dst2d = jnp.concatenate(
        [dst, jnp.full((pad,), N, jnp.int32)]).reshape(NCHUNKS, CH)

    zeros_h = jnp.zeros((RPT, H), jnp.float32)
    ones_h = jnp.ones((CH, H), jnp.float32)
    W2p = jnp.zeros((D, POUT), jnp.float32).at[:, :NCLS].set(W2)
    b0r = b0.reshape(1, D)
    b1r = b1.reshape(1, D)
    b2p = jnp.zeros((1, POUT), jnp.float32).at[0, :NCLS].set(b2)

    degp = _sc_deg(dst2d, zeros_h, ones_h)
    h0a, h0b, dinv = _tc1(degp, x, W0)
    agg0 = _sc_agg_feat(h0a, h0b, src2d, dst2d, zeros_h)
    h1a, h1b = _tc2(agg0, h0a, h0b, dinv, b0r, W1)
    agg1 = _sc_agg_feat(h1a, h1b, src2d, dst2d, zeros_h)
    h2, h2c = _tc3(agg1, h1a, h1b, dinv, b1r, W2p)
    agg2 = _sc_agg_edge(h2, h2c, src2d, dst2d, zeros_h)
    return _tc4(agg2, h2, dinv, b2p)
